# Initial kernel scaffold; baseline (speedup 1.0000x reference)
import jax
import jax.numpy as jnp
from jax.experimental import pallas as pl


def kernel(x, edge_index, edge_attr, batch, We0, be0, W1a0, b1a0, W1b0, b1b0, We1, be1, W1a1, b1a1, W1b1, b1b1, We2, be2, W1a2, b1a2, W1b2, b1b2, Wjk, bjk, Wc1, bc1, Wc2, bc2):
    raise NotImplementedError("write your pallas kernel here")


# speedup vs baseline: 3.3403x; 3.3403x over previous
import functools

import jax
import jax.numpy as jnp
from jax import lax
from jax.experimental import pallas as pl
from jax.experimental.pallas import tpu as pltpu
from jax.experimental.pallas import tpu_sc as plsc

F32 = jnp.float32

NC = 2
NS = 16
NW = NC * NS
LANES = 16

G = 64
H = 32
CLS = 2


@functools.lru_cache(maxsize=None)
def _make_msg_pass(N: int, d: int, E: int, B: int):
    EW = E // NW
    nchunk = EW // B
    RCH = 200
    NRC = N // RCH
    assert EW * NW == E and nchunk * B == EW and NRC * RCH == N
    assert B % 8 == 0 and B <= 128 and d % LANES == 0 and RCH % 8 == 0

    mesh = plsc.VectorSubcoreMesh(core_axis_name="c", subcore_axis_name="s")

    @functools.partial(
        pl.kernel,
        out_type=jax.ShapeDtypeStruct((NC, N, d), F32),
        mesh=mesh,
        compiler_params=pltpu.CompilerParams(use_tc_tiling_on_sc=False),
        scratch_types=[
            pltpu.VMEM((B,), jnp.int32),
            pltpu.VMEM((B,), jnp.int32),
            pltpu.VMEM((B, d), F32),
            pltpu.VMEM((B, d), F32),
            pltpu.VMEM_SHARED((N, d), F32),
            pltpu.SemaphoreType.DMA,
        ],
    )
    def msg_pass(h_hbm, e_hbm, src_hbm, dst_hbm, zr_hbm, out_hbm,
                 src_v, dst_v, hs_v, m_v, acc_sh, sem):
        c = lax.axis_index("c")
        s = lax.axis_index("s")
        wid = c * NS + s

        nrc_mine = NRC // NS + jnp.where(s < NRC % NS, 1, 0)

        def zero_chunk(k, carry):
            r0 = (s + k * NS) * RCH
            pltpu.sync_copy(zr_hbm, acc_sh.at[pl.ds(r0, RCH), :])
            return carry

        lax.fori_loop(0, nrc_mine, zero_chunk, 0)
        plsc.subcore_barrier()

        def chunk(i, carry):
            base = wid * EW + i * B
            pltpu.sync_copy(src_hbm.at[pl.ds(base, B)], src_v)
            pltpu.sync_copy(dst_hbm.at[pl.ds(base, B)], dst_v)
            gather = pltpu.async_copy(h_hbm.at[src_v], hs_v, sem)
            pltpu.sync_copy(e_hbm.at[pl.ds(base, B), :], m_v)
            gather.wait()

            def row(r, rcarry):
                for cc in range(d // LANES):
                    sl = pl.ds(cc * LANES, LANES)
                    m_v[r, sl] = jnp.maximum(hs_v[r, sl] + m_v[r, sl], 0.0)
                return rcarry

            lax.fori_loop(0, B, row, 0)
            pltpu.sync_copy(m_v, acc_sh.at[dst_v], add=True)
            return carry

        lax.fori_loop(0, nchunk, chunk, 0)
        plsc.subcore_barrier()

        def out_chunk(k, carry):
            r0 = (s + k * NS) * RCH
            pltpu.sync_copy(acc_sh.at[pl.ds(r0, RCH), :],
                            out_hbm.at[c, pl.ds(r0, RCH), :])
            return carry

        lax.fori_loop(0, nrc_mine, out_chunk, 0)

    return msg_pass


def _edge_transform(edge_attr, We0, be0, We1, be1, We2, be2, D):
    E = edge_attr.shape[0]
    BE = 4000
    nb = E // BE

    def body(a_ref, W0_ref, b0_ref, W1_ref, b1_ref, W2_ref, b2_ref,
             e0_ref, e1_ref, e2_ref):
        a = a_ref[...]
        e0_ref[...] = jnp.dot(a, W0_ref[...], preferred_element_type=F32) + b0_ref[...]
        e1_ref[...] = jnp.dot(a, W1_ref[...], preferred_element_type=F32) + b1_ref[...]
        e2_ref[...] = jnp.dot(a, W2_ref[...], preferred_element_type=F32) + b2_ref[...]

    DE = edge_attr.shape[1]
    full = lambda shape: pl.BlockSpec(shape, lambda i: (0, 0))
    return pl.pallas_call(
        body,
        grid=(nb,),
        in_specs=[
            pl.BlockSpec((BE, DE), lambda i: (i, 0)),
            full((DE, D)), full((1, D)),
            full((DE, H)), full((1, H)),
            full((DE, H)), full((1, H)),
        ],
        out_specs=[
            pl.BlockSpec((BE, D), lambda i: (i, 0)),
            pl.BlockSpec((BE, H), lambda i: (i, 0)),
            pl.BlockSpec((BE, H), lambda i: (i, 0)),
        ],
        out_shape=[
            jax.ShapeDtypeStruct((E, D), F32),
            jax.ShapeDtypeStruct((E, H), F32),
            jax.ShapeDtypeStruct((E, H), F32),
        ],
    )(edge_attr, We0, be0, We1, be1, We2, be2)


def _node_mlp(h, p0, p1, W1a, b1a, W1b, b1b):
    N, d = h.shape
    BN = 2000
    nb = N // BN

    def body(h_ref, p0_ref, p1_ref, Wa_ref, ba_ref, Wb_ref, bb_ref, o_ref):
        z = h_ref[...] + p0_ref[...] + p1_ref[...]
        z1 = jnp.maximum(
            jnp.dot(z, Wa_ref[...], preferred_element_type=F32) + ba_ref[...], 0.0)
        z2 = jnp.dot(z1, Wb_ref[...], preferred_element_type=F32) + bb_ref[...]
        o_ref[...] = jnp.maximum(z2, 0.0)

    full = lambda shape: pl.BlockSpec(shape, lambda i: (0, 0))
    return pl.pallas_call(
        body,
        grid=(nb,),
        in_specs=[
            pl.BlockSpec((BN, d), lambda i: (i, 0)),
            pl.BlockSpec((BN, d), lambda i: (i, 0)),
            pl.BlockSpec((BN, d), lambda i: (i, 0)),
            full((d, H)), full((1, H)),
            full((H, H)), full((1, H)),
        ],
        out_specs=pl.BlockSpec((BN, H), lambda i: (i, 0)),
        out_shape=jax.ShapeDtypeStruct((N, H), F32),
    )(h, p0, p1, W1a, b1a, W1b, b1b)


def _final_stage(h1, h2, p0, p1, batch3, W1a2, b1a2, W1b2, b1b2,
                 WjkX, bjkX, Wc1p, bc1p, Wc2p, bc2p):
    N = h1.shape[0]
    BN = 2000
    nb = N // BN

    def body(h1_ref, h2_ref, p0_ref, p1_ref, b_ref, Wa_ref, ba_ref,
             Wb_ref, bb_ref, Wjk_ref, bjk_ref, Wc1_ref, bc1_ref,
             Wc2_ref, bc2_ref, o_ref, sums_scr):
        pid = pl.program_id(0)

        @pl.when(pid == 0)
        def _():
            sums_scr[...] = jnp.zeros((G, 64), F32)

        z = h2_ref[...] + p0_ref[...] + p1_ref[...]
        z1 = jnp.maximum(
            jnp.dot(z, Wa_ref[...], preferred_element_type=F32) + ba_ref[...], 0.0)
        z2 = jnp.dot(z1, Wb_ref[...], preferred_element_type=F32) + bb_ref[...]
        h3 = jnp.maximum(z2, 0.0)

        cat = jnp.concatenate([h1_ref[...], h2_ref[...], h3], axis=1)
        o = jnp.dot(cat, Wjk_ref[...], preferred_element_type=F32) + bjk_ref[...]

        b = b_ref[0]
        oh = (lax.broadcasted_iota(jnp.int32, (G, BN), 0) == b).astype(F32)
        sums_scr[...] += jnp.dot(oh, o, preferred_element_type=F32)

        @pl.when(pid == nb - 1)
        def _():
            se = sums_scr[...]
            pooled = se[:, :H] / jnp.maximum(se[:, H:H + 1], 1.0)
            t = jnp.maximum(
                jnp.dot(pooled, Wc1_ref[...], preferred_element_type=F32)
                + bc1_ref[...], 0.0)
            logits = (jnp.dot(t, Wc2_ref[...], preferred_element_type=F32)
                      + bc2_ref[...])
            maskb = jnp.where(
                lax.broadcasted_iota(jnp.int32, (G, 128), 1) < CLS, 0.0, -1e30)
            logits = logits + maskb
            m = jnp.max(logits, axis=1, keepdims=True)
            lse = m + jnp.log(jnp.sum(jnp.exp(logits - m), axis=1, keepdims=True))
            o_ref[...] = logits - lse

    full = lambda shape: pl.BlockSpec(shape, lambda i: tuple(0 for _ in shape))
    return pl.pallas_call(
        body,
        grid=(nb,),
        in_specs=[
            pl.BlockSpec((BN, H), lambda i: (i, 0)),
            pl.BlockSpec((BN, H), lambda i: (i, 0)),
            pl.BlockSpec((BN, H), lambda i: (i, 0)),
            pl.BlockSpec((BN, H), lambda i: (i, 0)),
            pl.BlockSpec((1, 1, BN), lambda i: (i, 0, 0)),
            full((H, H)), full((1, H)),
            full((H, H)), full((1, H)),
            full((3 * H, 64)), full((1, 64)),
            full((H, 128)), full((1, 128)),
            full((128, 128)), full((1, 128)),
        ],
        out_specs=pl.BlockSpec((G, 128), lambda i: (0, 0)),
        out_shape=jax.ShapeDtypeStruct((G, 128), F32),
        scratch_shapes=[pltpu.VMEM((G, 64), F32)],
    )(h1, h2, p0, p1, batch3, W1a2, b1a2, W1b2, b1b2,
      WjkX, bjkX, Wc1p, bc1p, Wc2p, bc2p)


def kernel(x, edge_index, edge_attr, batch,
           We0, be0, W1a0, b1a0, W1b0, b1b0,
           We1, be1, W1a1, b1a1, W1b1, b1b1,
           We2, be2, W1a2, b1a2, W1b2, b1b2,
           Wjk, bjk, Wc1, bc1, Wc2, bc2):
    N, D = x.shape
    E = edge_index.shape[1]
    src = edge_index[0]
    dst = edge_index[1]

    r1 = lambda v: v.reshape(1, -1)

    e0, e1, e2 = _edge_transform(
        edge_attr, We0, r1(be0), We1, r1(be1), We2, r1(be2), D)

    B = 80
    zr_d = jnp.zeros((200, D), F32)
    zr_h = jnp.zeros((200, H), F32)

    p = _make_msg_pass(N, D, E, B)(x, e0, src, dst, zr_d)
    h1 = _node_mlp(x, p[0], p[1], W1a0, r1(b1a0), W1b0, r1(b1b0))

    mp32 = _make_msg_pass(N, H, E, B)
    p = mp32(h1, e1, src, dst, zr_h)
    h2 = _node_mlp(h1, p[0], p[1], W1a1, r1(b1a1), W1b1, r1(b1b1))

    p = mp32(h2, e2, src, dst, zr_h)

    WjkX = jnp.zeros((3 * H, 64), F32).at[:, :H].set(Wjk)
    bjkX = jnp.zeros((1, 64), F32).at[0, :H].set(bjk).at[0, H].set(1.0)
    Wc1p = jnp.zeros((H, 128), F32).at[:, :Wc1.shape[1]].set(Wc1)
    bc1p = jnp.zeros((1, 128), F32).at[0, :bc1.shape[0]].set(bc1)
    Wc2p = jnp.zeros((128, 128), F32).at[:Wc2.shape[0], :CLS].set(Wc2)
    bc2p = jnp.zeros((1, 128), F32).at[0, :CLS].set(bc2)

    BN = 2000
    batch3 = batch.reshape(N // BN, 1, BN)

    outp = _final_stage(h1, h2, p[0], p[1], batch3,
                        W1a2, r1(b1a2), W1b2, r1(b1b2),
                        WjkX, bjkX, Wc1p, bc1p, Wc2p, bc2p)
    return outp[:, :CLS]


# speedup vs baseline: 5.7503x; 1.7215x over previous
import functools

import jax
import jax.numpy as jnp
from jax import lax
from jax.experimental import pallas as pl
from jax.experimental.pallas import tpu as pltpu
from jax.experimental.pallas import tpu_sc as plsc

F32 = jnp.float32

NC = 2
NS = 16
NW = NC * NS
LANES = 16

G = 64
H = 32
CLS = 2


@functools.lru_cache(maxsize=None)
def _make_msg_pass(N: int, d: int, E: int, B: int):
    EW = E // NW
    nchunk = EW // B
    RCH = 200
    NRC = N // RCH
    assert EW * NW == E and nchunk * B == EW and NRC * RCH == N
    assert B % 8 == 0 and B <= 128 and d % LANES == 0 and RCH % 8 == 0

    mesh = plsc.VectorSubcoreMesh(core_axis_name="c", subcore_axis_name="s")
    RU = 4
    assert B % RU == 0

    @functools.partial(
        pl.kernel,
        out_type=jax.ShapeDtypeStruct((NC, N, d), F32),
        mesh=mesh,
        compiler_params=pltpu.CompilerParams(use_tc_tiling_on_sc=False),
        scratch_types=[
            pltpu.VMEM((nchunk, B), jnp.int32),
            pltpu.VMEM((nchunk, B), jnp.int32),
            pltpu.VMEM((2, B, d), F32),
            pltpu.VMEM((2, B, d), F32),
            pltpu.VMEM_SHARED((N, d), F32),
            pltpu.SemaphoreType.DMA,
            pltpu.SemaphoreType.DMA,
            pltpu.SemaphoreType.DMA,
            pltpu.SemaphoreType.DMA,
        ],
    )
    def msg_pass(h_hbm, e_hbm, src_hbm, dst_hbm, zr_hbm, out_hbm,
                 srcs_v, dsts_v, hs_v, m_v, acc_sh, gsem0, gsem1, esem0, esem1):
        c = lax.axis_index("c")
        s = lax.axis_index("s")
        wid = c * NS + s
        gsem = (gsem0, gsem1)
        esem = (esem0, esem1)

        nrc_mine = NRC // NS + jnp.where(s < NRC % NS, 1, 0)

        def zero_chunk(k, carry):
            r0 = (s + k * NS) * RCH
            pltpu.sync_copy(zr_hbm, acc_sh.at[pl.ds(r0, RCH), :])
            return carry

        lax.fori_loop(0, nrc_mine, zero_chunk, 0)

        pltpu.sync_copy(src_hbm.at[wid], srcs_v)
        pltpu.sync_copy(dst_hbm.at[wid], dsts_v)
        plsc.subcore_barrier()

        def start(j, b):
            pltpu.async_copy(h_hbm.at[srcs_v.at[j]], hs_v.at[b], gsem[b])
            pltpu.async_copy(e_hbm.at[pl.ds(wid * EW + j * B, B), :],
                             m_v.at[b], esem[b])

        def wait(j, b):
            pltpu.make_async_copy(h_hbm.at[srcs_v.at[j]], hs_v.at[b],
                                  gsem[b]).wait()
            pltpu.make_async_copy(e_hbm.at[pl.ds(wid * EW + j * B, B), :],
                                  m_v.at[b], esem[b]).wait()

        def compute(b):
            def rows(r0, rcarry):
                for rr in range(RU):
                    r = r0 * RU + rr
                    for cc in range(d // LANES):
                        sl = pl.ds(cc * LANES, LANES)
                        m_v[b, r, sl] = jnp.maximum(
                            hs_v[b, r, sl] + m_v[b, r, sl], 0.0)
                return rcarry

            lax.fori_loop(0, B // RU, rows, 0)

        def step(j, b):
            wait(j, b)
            compute(b)

            @pl.when(j + 2 < nchunk)
            def _():
                pltpu.async_copy(h_hbm.at[srcs_v.at[j + 2]], hs_v.at[b],
                                 gsem[b])

            pltpu.sync_copy(m_v.at[b], acc_sh.at[dsts_v.at[j]], add=True)

            @pl.when(j + 2 < nchunk)
            def _():
                pltpu.async_copy(
                    e_hbm.at[pl.ds(wid * EW + (j + 2) * B, B), :],
                    m_v.at[b], esem[b])

        start(0, 0)
        if nchunk > 1:
            start(1, 1)

        def pair(i, carry):
            step(2 * i, 0)
            step(2 * i + 1, 1)
            return carry

        lax.fori_loop(0, nchunk // 2, pair, 0)
        if nchunk % 2:
            step(nchunk - 1, (nchunk - 1) % 2)
        plsc.subcore_barrier()

        def out_chunk(k, carry):
            r0 = (s + k * NS) * RCH
            pltpu.sync_copy(acc_sh.at[pl.ds(r0, RCH), :],
                            out_hbm.at[c, pl.ds(r0, RCH), :])
            return carry

        lax.fori_loop(0, nrc_mine, out_chunk, 0)

    return msg_pass


def _edge_transform(edge_attr, We0, be0, We1, be1, We2, be2, D):
    E = edge_attr.shape[0]
    BE = 4000
    nb = E // BE

    def body(a_ref, W0_ref, b0_ref, W1_ref, b1_ref, W2_ref, b2_ref,
             e0_ref, e1_ref, e2_ref):
        a = a_ref[...]
        e0_ref[...] = jnp.dot(a, W0_ref[...], preferred_element_type=F32) + b0_ref[...]
        e1_ref[...] = jnp.dot(a, W1_ref[...], preferred_element_type=F32) + b1_ref[...]
        e2_ref[...] = jnp.dot(a, W2_ref[...], preferred_element_type=F32) + b2_ref[...]

    DE = edge_attr.shape[1]
    full = lambda shape: pl.BlockSpec(shape, lambda i: (0, 0))
    return pl.pallas_call(
        body,
        grid=(nb,),
        in_specs=[
            pl.BlockSpec((BE, DE), lambda i: (i, 0)),
            full((DE, D)), full((1, D)),
            full((DE, H)), full((1, H)),
            full((DE, H)), full((1, H)),
        ],
        out_specs=[
            pl.BlockSpec((BE, D), lambda i: (i, 0)),
            pl.BlockSpec((BE, H), lambda i: (i, 0)),
            pl.BlockSpec((BE, H), lambda i: (i, 0)),
        ],
        out_shape=[
            jax.ShapeDtypeStruct((E, D), F32),
            jax.ShapeDtypeStruct((E, H), F32),
            jax.ShapeDtypeStruct((E, H), F32),
        ],
    )(edge_attr, We0, be0, We1, be1, We2, be2)


def _node_mlp(h, p0, p1, W1a, b1a, W1b, b1b):
    N, d = h.shape
    BN = 2000
    nb = N // BN

    def body(h_ref, p0_ref, p1_ref, Wa_ref, ba_ref, Wb_ref, bb_ref, o_ref):
        z = h_ref[...] + p0_ref[...] + p1_ref[...]
        z1 = jnp.maximum(
            jnp.dot(z, Wa_ref[...], preferred_element_type=F32) + ba_ref[...], 0.0)
        z2 = jnp.dot(z1, Wb_ref[...], preferred_element_type=F32) + bb_ref[...]
        o_ref[...] = jnp.maximum(z2, 0.0)

    full = lambda shape: pl.BlockSpec(shape, lambda i: (0, 0))
    return pl.pallas_call(
        body,
        grid=(nb,),
        in_specs=[
            pl.BlockSpec((BN, d), lambda i: (i, 0)),
            pl.BlockSpec((BN, d), lambda i: (i, 0)),
            pl.BlockSpec((BN, d), lambda i: (i, 0)),
            full((d, H)), full((1, H)),
            full((H, H)), full((1, H)),
        ],
        out_specs=pl.BlockSpec((BN, H), lambda i: (i, 0)),
        out_shape=jax.ShapeDtypeStruct((N, H), F32),
    )(h, p0, p1, W1a, b1a, W1b, b1b)


def _final_stage(h1, h2, p0, p1, batch3, W1a2, b1a2, W1b2, b1b2,
                 WjkX, bjkX, Wc1p, bc1p, Wc2p, bc2p):
    N = h1.shape[0]
    BN = 2000
    nb = N // BN

    def body(h1_ref, h2_ref, p0_ref, p1_ref, b_ref, Wa_ref, ba_ref,
             Wb_ref, bb_ref, Wjk_ref, bjk_ref, Wc1_ref, bc1_ref,
             Wc2_ref, bc2_ref, o_ref, sums_scr):
        pid = pl.program_id(0)

        @pl.when(pid == 0)
        def _():
            sums_scr[...] = jnp.zeros((G, 64), F32)

        z = h2_ref[...] + p0_ref[...] + p1_ref[...]
        z1 = jnp.maximum(
            jnp.dot(z, Wa_ref[...], preferred_element_type=F32) + ba_ref[...], 0.0)
        z2 = jnp.dot(z1, Wb_ref[...], preferred_element_type=F32) + bb_ref[...]
        h3 = jnp.maximum(z2, 0.0)

        cat = jnp.concatenate([h1_ref[...], h2_ref[...], h3], axis=1)
        o = jnp.dot(cat, Wjk_ref[...], preferred_element_type=F32) + bjk_ref[...]

        b = b_ref[0]
        oh = (lax.broadcasted_iota(jnp.int32, (G, BN), 0) == b).astype(F32)
        sums_scr[...] += jnp.dot(oh, o, preferred_element_type=F32)

        @pl.when(pid == nb - 1)
        def _():
            se = sums_scr[...]
            pooled = se[:, :H] / jnp.maximum(se[:, H:H + 1], 1.0)
            t = jnp.maximum(
                jnp.dot(pooled, Wc1_ref[...], preferred_element_type=F32)
                + bc1_ref[...], 0.0)
            logits = (jnp.dot(t, Wc2_ref[...], preferred_element_type=F32)
                      + bc2_ref[...])
            maskb = jnp.where(
                lax.broadcasted_iota(jnp.int32, (G, 128), 1) < CLS, 0.0, -1e30)
            logits = logits + maskb
            m = jnp.max(logits, axis=1, keepdims=True)
            lse = m + jnp.log(jnp.sum(jnp.exp(logits - m), axis=1, keepdims=True))
            o_ref[...] = logits - lse

    full = lambda shape: pl.BlockSpec(shape, lambda i: tuple(0 for _ in shape))
    return pl.pallas_call(
        body,
        grid=(nb,),
        in_specs=[
            pl.BlockSpec((BN, H), lambda i: (i, 0)),
            pl.BlockSpec((BN, H), lambda i: (i, 0)),
            pl.BlockSpec((BN, H), lambda i: (i, 0)),
            pl.BlockSpec((BN, H), lambda i: (i, 0)),
            pl.BlockSpec((1, 1, BN), lambda i: (i, 0, 0)),
            full((H, H)), full((1, H)),
            full((H, H)), full((1, H)),
            full((3 * H, 64)), full((1, 64)),
            full((H, 128)), full((1, 128)),
            full((128, 128)), full((1, 128)),
        ],
        out_specs=pl.BlockSpec((G, 128), lambda i: (0, 0)),
        out_shape=jax.ShapeDtypeStruct((G, 128), F32),
        scratch_shapes=[pltpu.VMEM((G, 64), F32)],
    )(h1, h2, p0, p1, batch3, W1a2, b1a2, W1b2, b1b2,
      WjkX, bjkX, Wc1p, bc1p, Wc2p, bc2p)


def kernel(x, edge_index, edge_attr, batch,
           We0, be0, W1a0, b1a0, W1b0, b1b0,
           We1, be1, W1a1, b1a1, W1b1, b1b1,
           We2, be2, W1a2, b1a2, W1b2, b1b2,
           Wjk, bjk, Wc1, bc1, Wc2, bc2):
    N, D = x.shape
    E = edge_index.shape[1]
    B0, B1 = 40, 80
    src0 = edge_index[0].reshape(NW, E // (NW * B0), B0)
    dst0 = edge_index[1].reshape(NW, E // (NW * B0), B0)
    src1 = edge_index[0].reshape(NW, E // (NW * B1), B1)
    dst1 = edge_index[1].reshape(NW, E // (NW * B1), B1)

    r1 = lambda v: v.reshape(1, -1)

    e0, e1, e2 = _edge_transform(
        edge_attr, We0, r1(be0), We1, r1(be1), We2, r1(be2), D)

    zr_d = jnp.zeros((200, D), F32)
    zr_h = jnp.zeros((200, H), F32)

    p = _make_msg_pass(N, D, E, B0)(x, e0, src0, dst0, zr_d)
    h1 = _node_mlp(x, p[0], p[1], W1a0, r1(b1a0), W1b0, r1(b1b0))

    mp32 = _make_msg_pass(N, H, E, B1)
    p = mp32(h1, e1, src1, dst1, zr_h)
    h2 = _node_mlp(h1, p[0], p[1], W1a1, r1(b1a1), W1b1, r1(b1b1))

    p = mp32(h2, e2, src1, dst1, zr_h)

    WjkX = jnp.zeros((3 * H, 64), F32).at[:, :H].set(Wjk)
    bjkX = jnp.zeros((1, 64), F32).at[0, :H].set(bjk).at[0, H].set(1.0)
    Wc1p = jnp.zeros((H, 128), F32).at[:, :Wc1.shape[1]].set(Wc1)
    bc1p = jnp.zeros((1, 128), F32).at[0, :bc1.shape[0]].set(bc1)
    Wc2p = jnp.zeros((128, 128), F32).at[:Wc2.shape[0], :CLS].set(Wc2)
    bc2p = jnp.zeros((1, 128), F32).at[0, :CLS].set(bc2)

    BN = 2000
    batch3 = batch.reshape(N // BN, 1, BN)

    outp = _final_stage(h1, h2, p[0], p[1], batch3,
                        W1a2, r1(b1a2), W1b2, r1(b1b2),
                        WjkX, bjkX, Wc1p, bc1p, Wc2p, bc2p)
    return outp[:, :CLS]


# speedup vs baseline: 5.8479x; 1.0170x over previous
import functools

import jax
import jax.numpy as jnp
from jax import lax
from jax.experimental import pallas as pl
from jax.experimental.pallas import tpu as pltpu
from jax.experimental.pallas import tpu_sc as plsc

F32 = jnp.float32

NC = 2
NS = 16
NW = NC * NS
LANES = 16

G = 64
H = 32
CLS = 2


@functools.lru_cache(maxsize=None)
def _make_msg_pass(N: int, d: int, E: int, B: int):
    EW = E // NW
    nchunk = EW // B
    RCH = 200
    NRC = N // RCH
    assert EW * NW == E and nchunk * B == EW and NRC * RCH == N
    assert B % 8 == 0 and B <= 128 and d % LANES == 0 and RCH % 8 == 0

    mesh = plsc.VectorSubcoreMesh(core_axis_name="c", subcore_axis_name="s")
    RU = 4
    assert B % RU == 0

    @functools.partial(
        pl.kernel,
        out_type=jax.ShapeDtypeStruct((NC, N, d), F32),
        mesh=mesh,
        compiler_params=pltpu.CompilerParams(use_tc_tiling_on_sc=False),
        scratch_types=[
            pltpu.VMEM((nchunk, B), jnp.int32),
            pltpu.VMEM((nchunk, B), jnp.int32),
            pltpu.VMEM((2, B, d), F32),
            pltpu.VMEM((2, B, d), F32),
            pltpu.VMEM((2, B, d), F32),
            pltpu.VMEM_SHARED((N, d), F32),
            pltpu.SemaphoreType.DMA,
            pltpu.SemaphoreType.DMA,
            pltpu.SemaphoreType.DMA,
            pltpu.SemaphoreType.DMA,
            pltpu.SemaphoreType.DMA,
            pltpu.SemaphoreType.DMA,
        ],
    )
    def msg_pass(h_hbm, e_hbm, src_hbm, dst_hbm, zr_hbm, out_hbm,
                 srcs_v, dsts_v, hs_v, e_v, m_v, acc_sh,
                 gsem0, gsem1, esem0, esem1, ssem0, ssem1):
        c = lax.axis_index("c")
        s = lax.axis_index("s")
        wid = c * NS + s
        gsem = (gsem0, gsem1)
        esem = (esem0, esem1)
        ssem = (ssem0, ssem1)

        nrc_mine = NRC // NS + jnp.where(s < NRC % NS, 1, 0)

        def zero_chunk(k, carry):
            r0 = (s + k * NS) * RCH
            pltpu.sync_copy(zr_hbm, acc_sh.at[pl.ds(r0, RCH), :])
            return carry

        lax.fori_loop(0, nrc_mine, zero_chunk, 0)

        pltpu.sync_copy(src_hbm.at[wid], srcs_v)
        pltpu.sync_copy(dst_hbm.at[wid], dsts_v)
        plsc.subcore_barrier()

        def start(j, b):
            pltpu.async_copy(h_hbm.at[srcs_v.at[j]], hs_v.at[b], gsem[b])
            pltpu.async_copy(e_hbm.at[pl.ds(wid * EW + j * B, B), :],
                             e_v.at[b], esem[b])

        def wait_in(j, b):
            pltpu.make_async_copy(h_hbm.at[srcs_v.at[j]], hs_v.at[b],
                                  gsem[b]).wait()
            pltpu.make_async_copy(e_hbm.at[pl.ds(wid * EW + j * B, B), :],
                                  e_v.at[b], esem[b]).wait()

        def wait_scat(j, b):
            pltpu.make_async_copy(m_v.at[b], acc_sh.at[dsts_v.at[j]],
                                  ssem[b]).wait()

        def compute(b):
            def rows(r0, rcarry):
                for rr in range(RU):
                    r = r0 * RU + rr
                    for cc in range(d // LANES):
                        sl = pl.ds(cc * LANES, LANES)
                        m_v[b, r, sl] = jnp.maximum(
                            hs_v[b, r, sl] + e_v[b, r, sl], 0.0)
                return rcarry

            lax.fori_loop(0, B // RU, rows, 0)

        def step(j, b):
            wait_in(j, b)

            @pl.when(j >= 2)
            def _():
                wait_scat(j - 2, b)

            compute(b)

            @pl.when(j + 2 < nchunk)
            def _():
                start(j + 2, b)

            pltpu.async_copy(m_v.at[b], acc_sh.at[dsts_v.at[j]], ssem[b],
                             add=True)

        start(0, 0)
        if nchunk > 1:
            start(1, 1)

        def pair(i, carry):
            step(2 * i, 0)
            step(2 * i + 1, 1)
            return carry

        lax.fori_loop(0, nchunk // 2, pair, 0)
        if nchunk % 2:
            step(nchunk - 1, (nchunk - 1) % 2)
        if nchunk >= 2:
            wait_scat(nchunk - 2, (nchunk - 2) % 2)
        wait_scat(nchunk - 1, (nchunk - 1) % 2)
        plsc.subcore_barrier()

        def out_chunk(k, carry):
            r0 = (s + k * NS) * RCH
            pltpu.sync_copy(acc_sh.at[pl.ds(r0, RCH), :],
                            out_hbm.at[c, pl.ds(r0, RCH), :])
            return carry

        lax.fori_loop(0, nrc_mine, out_chunk, 0)

    return msg_pass


def _edge_transform(edge_attr, We0, be0, We1, be1, We2, be2, D):
    E = edge_attr.shape[0]
    BE = 4000
    nb = E // BE

    def body(a_ref, W0_ref, b0_ref, W1_ref, b1_ref, W2_ref, b2_ref,
             e0_ref, e1_ref, e2_ref):
        a = a_ref[...]
        e0_ref[...] = jnp.dot(a, W0_ref[...], preferred_element_type=F32) + b0_ref[...]
        e1_ref[...] = jnp.dot(a, W1_ref[...], preferred_element_type=F32) + b1_ref[...]
        e2_ref[...] = jnp.dot(a, W2_ref[...], preferred_element_type=F32) + b2_ref[...]

    DE = edge_attr.shape[1]
    full = lambda shape: pl.BlockSpec(shape, lambda i: (0, 0))
    return pl.pallas_call(
        body,
        grid=(nb,),
        in_specs=[
            pl.BlockSpec((BE, DE), lambda i: (i, 0)),
            full((DE, D)), full((1, D)),
            full((DE, H)), full((1, H)),
            full((DE, H)), full((1, H)),
        ],
        out_specs=[
            pl.BlockSpec((BE, D), lambda i: (i, 0)),
            pl.BlockSpec((BE, H), lambda i: (i, 0)),
            pl.BlockSpec((BE, H), lambda i: (i, 0)),
        ],
        out_shape=[
            jax.ShapeDtypeStruct((E, D), F32),
            jax.ShapeDtypeStruct((E, H), F32),
            jax.ShapeDtypeStruct((E, H), F32),
        ],
    )(edge_attr, We0, be0, We1, be1, We2, be2)


def _node_mlp(h, p0, p1, W1a, b1a, W1b, b1b):
    N, d = h.shape
    BN = 2000
    nb = N // BN

    def body(h_ref, p0_ref, p1_ref, Wa_ref, ba_ref, Wb_ref, bb_ref, o_ref):
        z = h_ref[...] + p0_ref[...] + p1_ref[...]
        z1 = jnp.maximum(
            jnp.dot(z, Wa_ref[...], preferred_element_type=F32) + ba_ref[...], 0.0)
        z2 = jnp.dot(z1, Wb_ref[...], preferred_element_type=F32) + bb_ref[...]
        o_ref[...] = jnp.maximum(z2, 0.0)

    full = lambda shape: pl.BlockSpec(shape, lambda i: (0, 0))
    return pl.pallas_call(
        body,
        grid=(nb,),
        in_specs=[
            pl.BlockSpec((BN, d), lambda i: (i, 0)),
            pl.BlockSpec((BN, d), lambda i: (i, 0)),
            pl.BlockSpec((BN, d), lambda i: (i, 0)),
            full((d, H)), full((1, H)),
            full((H, H)), full((1, H)),
        ],
        out_specs=pl.BlockSpec((BN, H), lambda i: (i, 0)),
        out_shape=jax.ShapeDtypeStruct((N, H), F32),
    )(h, p0, p1, W1a, b1a, W1b, b1b)


def _final_stage(h1, h2, p0, p1, batch3, W1a2, b1a2, W1b2, b1b2,
                 WjkX, bjkX, Wc1p, bc1p, Wc2p, bc2p):
    N = h1.shape[0]
    BN = 2000
    nb = N // BN

    def body(h1_ref, h2_ref, p0_ref, p1_ref, b_ref, Wa_ref, ba_ref,
             Wb_ref, bb_ref, Wjk_ref, bjk_ref, Wc1_ref, bc1_ref,
             Wc2_ref, bc2_ref, o_ref, sums_scr):
        pid = pl.program_id(0)

        @pl.when(pid == 0)
        def _():
            sums_scr[...] = jnp.zeros((G, 64), F32)

        z = h2_ref[...] + p0_ref[...] + p1_ref[...]
        z1 = jnp.maximum(
            jnp.dot(z, Wa_ref[...], preferred_element_type=F32) + ba_ref[...], 0.0)
        z2 = jnp.dot(z1, Wb_ref[...], preferred_element_type=F32) + bb_ref[...]
        h3 = jnp.maximum(z2, 0.0)

        cat = jnp.concatenate([h1_ref[...], h2_ref[...], h3], axis=1)
        o = jnp.dot(cat, Wjk_ref[...], preferred_element_type=F32) + bjk_ref[...]

        b = b_ref[0]
        oh = (lax.broadcasted_iota(jnp.int32, (G, BN), 0) == b).astype(F32)
        sums_scr[...] += jnp.dot(oh, o, preferred_element_type=F32)

        @pl.when(pid == nb - 1)
        def _():
            se = sums_scr[...]
            pooled = se[:, :H] / jnp.maximum(se[:, H:H + 1], 1.0)
            t = jnp.maximum(
                jnp.dot(pooled, Wc1_ref[...], preferred_element_type=F32)
                + bc1_ref[...], 0.0)
            logits = (jnp.dot(t, Wc2_ref[...], preferred_element_type=F32)
                      + bc2_ref[...])
            maskb = jnp.where(
                lax.broadcasted_iota(jnp.int32, (G, 128), 1) < CLS, 0.0, -1e30)
            logits = logits + maskb
            m = jnp.max(logits, axis=1, keepdims=True)
            lse = m + jnp.log(jnp.sum(jnp.exp(logits - m), axis=1, keepdims=True))
            o_ref[...] = logits - lse

    full = lambda shape: pl.BlockSpec(shape, lambda i: tuple(0 for _ in shape))
    return pl.pallas_call(
        body,
        grid=(nb,),
        in_specs=[
            pl.BlockSpec((BN, H), lambda i: (i, 0)),
            pl.BlockSpec((BN, H), lambda i: (i, 0)),
            pl.BlockSpec((BN, H), lambda i: (i, 0)),
            pl.BlockSpec((BN, H), lambda i: (i, 0)),
            pl.BlockSpec((1, 1, BN), lambda i: (i, 0, 0)),
            full((H, H)), full((1, H)),
            full((H, H)), full((1, H)),
            full((3 * H, 64)), full((1, 64)),
            full((H, 128)), full((1, 128)),
            full((128, 128)), full((1, 128)),
        ],
        out_specs=pl.BlockSpec((G, 128), lambda i: (0, 0)),
        out_shape=jax.ShapeDtypeStruct((G, 128), F32),
        scratch_shapes=[pltpu.VMEM((G, 64), F32)],
    )(h1, h2, p0, p1, batch3, W1a2, b1a2, W1b2, b1b2,
      WjkX, bjkX, Wc1p, bc1p, Wc2p, bc2p)


def kernel(x, edge_index, edge_attr, batch,
           We0, be0, W1a0, b1a0, W1b0, b1b0,
           We1, be1, W1a1, b1a1, W1b1, b1b1,
           We2, be2, W1a2, b1a2, W1b2, b1b2,
           Wjk, bjk, Wc1, bc1, Wc2, bc2):
    N, D = x.shape
    E = edge_index.shape[1]
    B0, B1 = 40, 80
    src0 = edge_index[0].reshape(NW, E // (NW * B0), B0)
    dst0 = edge_index[1].reshape(NW, E // (NW * B0), B0)
    src1 = edge_index[0].reshape(NW, E // (NW * B1), B1)
    dst1 = edge_index[1].reshape(NW, E // (NW * B1), B1)

    r1 = lambda v: v.reshape(1, -1)

    e0, e1, e2 = _edge_transform(
        edge_attr, We0, r1(be0), We1, r1(be1), We2, r1(be2), D)

    zr_d = jnp.zeros((200, D), F32)
    zr_h = jnp.zeros((200, H), F32)

    p = _make_msg_pass(N, D, E, B0)(x, e0, src0, dst0, zr_d)
    h1 = _node_mlp(x, p[0], p[1], W1a0, r1(b1a0), W1b0, r1(b1b0))

    mp32 = _make_msg_pass(N, H, E, B1)
    p = mp32(h1, e1, src1, dst1, zr_h)
    h2 = _node_mlp(h1, p[0], p[1], W1a1, r1(b1a1), W1b1, r1(b1b1))

    p = mp32(h2, e2, src1, dst1, zr_h)

    WjkX = jnp.zeros((3 * H, 64), F32).at[:, :H].set(Wjk)
    bjkX = jnp.zeros((1, 64), F32).at[0, :H].set(bjk).at[0, H].set(1.0)
    Wc1p = jnp.zeros((H, 128), F32).at[:, :Wc1.shape[1]].set(Wc1)
    bc1p = jnp.zeros((1, 128), F32).at[0, :bc1.shape[0]].set(bc1)
    Wc2p = jnp.zeros((128, 128), F32).at[:Wc2.shape[0], :CLS].set(Wc2)
    bc2p = jnp.zeros((1, 128), F32).at[0, :CLS].set(bc2)

    BN = 2000
    batch3 = batch.reshape(N // BN, 1, BN)

    outp = _final_stage(h1, h2, p[0], p[1], batch3,
                        W1a2, r1(b1a2), W1b2, r1(b1b2),
                        WjkX, bjkX, Wc1p, bc1p, Wc2p, bc2p)
    return outp[:, :CLS]


# speedup vs baseline: 7.4764x; 1.2785x over previous
import functools

import jax
import jax.numpy as jnp
from jax import lax
from jax.experimental import pallas as pl
from jax.experimental.pallas import tpu as pltpu
from jax.experimental.pallas import tpu_sc as plsc

F32 = jnp.float32

NC = 2
NS = 16
NW = NC * NS
LANES = 16

G = 64
H = 32
CLS = 2


@functools.lru_cache(maxsize=None)
def _make_msg_pass(N: int, d: int, E: int, B: int):
    EW = E // NW
    nchunk = EW // B
    RCH = 200
    NRC = N // RCH
    assert EW * NW == E and nchunk * B == EW and NRC * RCH == N
    assert B % 8 == 0 and B <= 128 and d % LANES == 0 and RCH % 8 == 0

    mesh = plsc.VectorSubcoreMesh(core_axis_name="c", subcore_axis_name="s")
    RU = 4
    assert B % RU == 0 and (RU * d) % 128 == 0
    EB = B * d // 128
    ERW = EW * d // 128

    @functools.partial(
        pl.kernel,
        out_type=jax.ShapeDtypeStruct((NC, N, d), F32),
        mesh=mesh,
        compiler_params=pltpu.CompilerParams(use_tc_tiling_on_sc=False),
        scratch_types=[
            pltpu.VMEM((nchunk, B), jnp.int32),
            pltpu.VMEM((nchunk, B), jnp.int32),
            pltpu.VMEM((2, B, d), F32),
            pltpu.VMEM((2, EB, 128), F32),
            pltpu.VMEM((2, B, d), F32),
            pltpu.VMEM_SHARED((N, d), F32),
            pltpu.SemaphoreType.DMA,
            pltpu.SemaphoreType.DMA,
            pltpu.SemaphoreType.DMA,
            pltpu.SemaphoreType.DMA,
            pltpu.SemaphoreType.DMA,
            pltpu.SemaphoreType.DMA,
        ],
    )
    def msg_pass(h_hbm, e_hbm, src_hbm, dst_hbm, zr_hbm, out_hbm,
                 srcs_v, dsts_v, hs_v, e_v, m_v, acc_sh,
                 gsem0, gsem1, esem0, esem1, ssem0, ssem1):
        c = lax.axis_index("c")
        s = lax.axis_index("s")
        wid = c * NS + s
        gsem = (gsem0, gsem1)
        esem = (esem0, esem1)
        ssem = (ssem0, ssem1)

        nrc_mine = NRC // NS + jnp.where(s < NRC % NS, 1, 0)

        def zero_chunk(k, carry):
            r0 = (s + k * NS) * RCH
            pltpu.sync_copy(zr_hbm, acc_sh.at[pl.ds(r0, RCH), :])
            return carry

        lax.fori_loop(0, nrc_mine, zero_chunk, 0)

        pltpu.sync_copy(src_hbm.at[wid], srcs_v)
        pltpu.sync_copy(dst_hbm.at[wid], dsts_v)
        plsc.subcore_barrier()

        def start(j, b):
            pltpu.async_copy(h_hbm.at[srcs_v.at[j]], hs_v.at[b], gsem[b])
            pltpu.async_copy(e_hbm.at[pl.ds(wid * ERW + j * EB, EB), :],
                             e_v.at[b], esem[b])

        def wait_in(j, b):
            pltpu.make_async_copy(h_hbm.at[srcs_v.at[j]], hs_v.at[b],
                                  gsem[b]).wait()
            pltpu.make_async_copy(e_hbm.at[pl.ds(wid * ERW + j * EB, EB), :],
                                  e_v.at[b], esem[b]).wait()

        def wait_scat(j, b):
            pltpu.make_async_copy(m_v.at[b], acc_sh.at[dsts_v.at[j]],
                                  ssem[b]).wait()

        def compute(b):
            def rows(r0, rcarry):
                er0 = r0 * (RU * d // 128)
                for rr in range(RU):
                    r = r0 * RU + rr
                    er = er0 + (rr * d) // 128
                    el0 = (rr * d) % 128
                    for cc in range(d // LANES):
                        sl = pl.ds(cc * LANES, LANES)
                        sle = pl.ds(el0 + cc * LANES, LANES)
                        m_v[b, r, sl] = jnp.maximum(
                            hs_v[b, r, sl] + e_v[b, er, sle], 0.0)
                return rcarry

            lax.fori_loop(0, B // RU, rows, 0)

        def step(j, b):
            wait_in(j, b)

            @pl.when(j >= 2)
            def _():
                wait_scat(j - 2, b)

            compute(b)

            @pl.when(j + 2 < nchunk)
            def _():
                start(j + 2, b)

            pltpu.async_copy(m_v.at[b], acc_sh.at[dsts_v.at[j]], ssem[b],
                             add=True)

        start(0, 0)
        if nchunk > 1:
            start(1, 1)

        def pair(i, carry):
            step(2 * i, 0)
            step(2 * i + 1, 1)
            return carry

        lax.fori_loop(0, nchunk // 2, pair, 0)
        if nchunk % 2:
            step(nchunk - 1, (nchunk - 1) % 2)
        if nchunk >= 2:
            wait_scat(nchunk - 2, (nchunk - 2) % 2)
        wait_scat(nchunk - 1, (nchunk - 1) % 2)
        plsc.subcore_barrier()

        def out_chunk(k, carry):
            r0 = (s + k * NS) * RCH
            pltpu.sync_copy(acc_sh.at[pl.ds(r0, RCH), :],
                            out_hbm.at[c, pl.ds(r0, RCH), :])
            return carry

        lax.fori_loop(0, nrc_mine, out_chunk, 0)

    return msg_pass


_DNT = (((0,), (0,)), ((), ()))


def _edge_transform_e0(ea_t, We0, be0, D):
    DE, E = ea_t.shape
    BE = 3200
    nb = E // BE

    def body(a_ref, W0_ref, b0_ref, e0_ref):
        a = a_ref[...]
        e0_ref[...] = lax.dot_general(
            a, W0_ref[...], _DNT, preferred_element_type=F32) + b0_ref[...]

    full = lambda shape: pl.BlockSpec(shape, lambda i: (0, 0))
    return pl.pallas_call(
        body,
        grid=(nb,),
        in_specs=[
            pl.BlockSpec((DE, BE), lambda i: (0, i)),
            full((DE, D)), full((1, D)),
        ],
        out_specs=pl.BlockSpec((BE, D), lambda i: (i, 0)),
        out_shape=jax.ShapeDtypeStruct((E, D), F32),
    )(ea_t, We0, be0)


def _edge_transform_e12(ea_t, We1, be1, We2, be2):
    DE, E = ea_t.shape
    BE = 3200
    nb = E // BE
    R = BE * H // 128

    def body(a_ref, W1_ref, b1_ref, W2_ref, b2_ref, e1_ref, e2_ref):
        a = a_ref[...]
        r1 = lax.dot_general(
            a, W1_ref[...], _DNT, preferred_element_type=F32) + b1_ref[...]
        r2 = lax.dot_general(
            a, W2_ref[...], _DNT, preferred_element_type=F32) + b2_ref[...]
        e1_ref[...] = jnp.concatenate(
            [r1[i * R:(i + 1) * R, :] for i in range(128 // H)], axis=1)
        e2_ref[...] = jnp.concatenate(
            [r2[i * R:(i + 1) * R, :] for i in range(128 // H)], axis=1)

    full = lambda shape: pl.BlockSpec(shape, lambda i: (0, 0))
    return pl.pallas_call(
        body,
        grid=(nb,),
        in_specs=[
            pl.BlockSpec((DE, BE), lambda i: (0, i)),
            full((DE, H)), full((1, H)),
            full((DE, H)), full((1, H)),
        ],
        out_specs=[
            pl.BlockSpec((R, 128), lambda i: (i, 0)),
            pl.BlockSpec((R, 128), lambda i: (i, 0)),
        ],
        out_shape=[
            jax.ShapeDtypeStruct((E * H // 128, 128), F32),
            jax.ShapeDtypeStruct((E * H // 128, 128), F32),
        ],
    )(ea_t, We1, be1, We2, be2)


def _node_mlp(h, p0, p1, W1a, b1a, W1b, b1b):
    N, d = h.shape
    BN = 2000
    nb = N // BN

    def body(h_ref, p0_ref, p1_ref, Wa_ref, ba_ref, Wb_ref, bb_ref, o_ref):
        z = h_ref[...] + p0_ref[...] + p1_ref[...]
        z1 = jnp.maximum(
            jnp.dot(z, Wa_ref[...], preferred_element_type=F32) + ba_ref[...], 0.0)
        z2 = jnp.dot(z1, Wb_ref[...], preferred_element_type=F32) + bb_ref[...]
        o_ref[...] = jnp.maximum(z2, 0.0)

    full = lambda shape: pl.BlockSpec(shape, lambda i: (0, 0))
    return pl.pallas_call(
        body,
        grid=(nb,),
        in_specs=[
            pl.BlockSpec((BN, d), lambda i: (i, 0)),
            pl.BlockSpec((BN, d), lambda i: (i, 0)),
            pl.BlockSpec((BN, d), lambda i: (i, 0)),
            full((d, H)), full((1, H)),
            full((H, H)), full((1, H)),
        ],
        out_specs=pl.BlockSpec((BN, H), lambda i: (i, 0)),
        out_shape=jax.ShapeDtypeStruct((N, H), F32),
    )(h, p0, p1, W1a, b1a, W1b, b1b)


def _final_stage(h1, h2, p0, p1, batch3, W1a2, b1a2, W1b2, b1b2,
                 WjkX, bjkX, Wc1p, bc1p, Wc2p, bc2p):
    N = h1.shape[0]
    BN = 2000
    nb = N // BN

    def body(h1_ref, h2_ref, p0_ref, p1_ref, b_ref, Wa_ref, ba_ref,
             Wb_ref, bb_ref, Wjk_ref, bjk_ref, Wc1_ref, bc1_ref,
             Wc2_ref, bc2_ref, o_ref, sums_scr):
        pid = pl.program_id(0)

        @pl.when(pid == 0)
        def _():
            sums_scr[...] = jnp.zeros((G, 64), F32)

        z = h2_ref[...] + p0_ref[...] + p1_ref[...]
        z1 = jnp.maximum(
            jnp.dot(z, Wa_ref[...], preferred_element_type=F32) + ba_ref[...], 0.0)
        z2 = jnp.dot(z1, Wb_ref[...], preferred_element_type=F32) + bb_ref[...]
        h3 = jnp.maximum(z2, 0.0)

        cat = jnp.concatenate([h1_ref[...], h2_ref[...], h3], axis=1)
        o = jnp.dot(cat, Wjk_ref[...], preferred_element_type=F32) + bjk_ref[...]

        b = b_ref[0]
        oh = (lax.broadcasted_iota(jnp.int32, (G, BN), 0) == b).astype(F32)
        sums_scr[...] += jnp.dot(oh, o, preferred_element_type=F32)

        @pl.when(pid == nb - 1)
        def _():
            se = sums_scr[...]
            pooled = se[:, :H] / jnp.maximum(se[:, H:H + 1], 1.0)
            t = jnp.maximum(
                jnp.dot(pooled, Wc1_ref[...], preferred_element_type=F32)
                + bc1_ref[...], 0.0)
            logits = (jnp.dot(t, Wc2_ref[...], preferred_element_type=F32)
                      + bc2_ref[...])
            maskb = jnp.where(
                lax.broadcasted_iota(jnp.int32, (G, 128), 1) < CLS, 0.0, -1e30)
            logits = logits + maskb
            m = jnp.max(logits, axis=1, keepdims=True)
            lse = m + jnp.log(jnp.sum(jnp.exp(logits - m), axis=1, keepdims=True))
            o_ref[...] = logits - lse

    full = lambda shape: pl.BlockSpec(shape, lambda i: tuple(0 for _ in shape))
    return pl.pallas_call(
        body,
        grid=(nb,),
        in_specs=[
            pl.BlockSpec((BN, H), lambda i: (i, 0)),
            pl.BlockSpec((BN, H), lambda i: (i, 0)),
            pl.BlockSpec((BN, H), lambda i: (i, 0)),
            pl.BlockSpec((BN, H), lambda i: (i, 0)),
            pl.BlockSpec((1, 1, BN), lambda i: (i, 0, 0)),
            full((H, H)), full((1, H)),
            full((H, H)), full((1, H)),
            full((3 * H, 64)), full((1, 64)),
            full((H, 128)), full((1, 128)),
            full((128, 128)), full((1, 128)),
        ],
        out_specs=pl.BlockSpec((G, 128), lambda i: (0, 0)),
        out_shape=jax.ShapeDtypeStruct((G, 128), F32),
        scratch_shapes=[pltpu.VMEM((G, 64), F32)],
    )(h1, h2, p0, p1, batch3, W1a2, b1a2, W1b2, b1b2,
      WjkX, bjkX, Wc1p, bc1p, Wc2p, bc2p)


def kernel(x, edge_index, edge_attr, batch,
           We0, be0, W1a0, b1a0, W1b0, b1b0,
           We1, be1, W1a1, b1a1, W1b1, b1b1,
           We2, be2, W1a2, b1a2, W1b2, b1b2,
           Wjk, bjk, Wc1, bc1, Wc2, bc2):
    N, D = x.shape
    E = edge_index.shape[1]
    B0, B1 = 40, 80
    src0 = edge_index[0].reshape(NW, E // (NW * B0), B0)
    dst0 = edge_index[1].reshape(NW, E // (NW * B0), B0)

    BE, R = 3200, 3200 * H // 128
    idx = jnp.arange(E, dtype=jnp.int32)
    q, jj = idx // 4, idx % 4
    orig = (q // R) * BE + jj * R + (q % R)
    src1 = edge_index[0][orig].reshape(NW, E // (NW * B1), B1)
    dst1 = edge_index[1][orig].reshape(NW, E // (NW * B1), B1)

    r1 = lambda v: v.reshape(1, -1)
    ea_t = edge_attr.T

    e0 = _edge_transform_e0(ea_t, We0, r1(be0), D)

    zr_d = jnp.zeros((200, D), F32)
    zr_h = jnp.zeros((200, H), F32)

    p = _make_msg_pass(N, D, E, B0)(x, e0, src0, dst0, zr_d)
    e1r, e2r = _edge_transform_e12(ea_t, We1, r1(be1), We2, r1(be2))
    h1 = _node_mlp(x, p[0], p[1], W1a0, r1(b1a0), W1b0, r1(b1b0))

    mp32 = _make_msg_pass(N, H, E, B1)
    p = mp32(h1, e1r, src1, dst1, zr_h)
    h2 = _node_mlp(h1, p[0], p[1], W1a1, r1(b1a1), W1b1, r1(b1b1))

    p = mp32(h2, e2r, src1, dst1, zr_h)

    WjkX = jnp.zeros((3 * H, 64), F32).at[:, :H].set(Wjk)
    bjkX = jnp.zeros((1, 64), F32).at[0, :H].set(bjk).at[0, H].set(1.0)
    Wc1p = jnp.zeros((H, 128), F32).at[:, :Wc1.shape[1]].set(Wc1)
    bc1p = jnp.zeros((1, 128), F32).at[0, :bc1.shape[0]].set(bc1)
    Wc2p = jnp.zeros((128, 128), F32).at[:Wc2.shape[0], :CLS].set(Wc2)
    bc2p = jnp.zeros((1, 128), F32).at[0, :CLS].set(bc2)

    BN = 2000
    batch3 = batch.reshape(N // BN, 1, BN)

    outp = _final_stage(h1, h2, p[0], p[1], batch3,
                        W1a2, r1(b1a2), W1b2, r1(b1b2),
                        WjkX, bjkX, Wc1p, bc1p, Wc2p, bc2p)
    return outp[:, :CLS]


# speedup vs baseline: 7.5790x; 1.0137x over previous
import functools

import jax
import jax.numpy as jnp
from jax import lax
from jax.experimental import pallas as pl
from jax.experimental.pallas import tpu as pltpu
from jax.experimental.pallas import tpu_sc as plsc

F32 = jnp.float32

NC = 2
NS = 16
NW = NC * NS
LANES = 16

G = 64
H = 32
CLS = 2


@functools.lru_cache(maxsize=None)
def _make_msg_pass(N: int, d: int, E: int, B: int):
    EW = E // NW
    nchunk = EW // B
    RCH = 200
    NRC = N // RCH
    assert EW * NW == E and nchunk * B == EW and NRC * RCH == N
    assert B % 8 == 0 and B <= 128 and d % LANES == 0 and RCH % 8 == 0

    mesh = plsc.VectorSubcoreMesh(core_axis_name="c", subcore_axis_name="s")
    RU = 4
    assert B % RU == 0 and (RU * d) % 128 == 0
    EB = B * d // 128
    ERW = EW * d // 128
    SP = N * d * 4 <= 2 * 1024 * 1024

    scratch = [
        pltpu.VMEM((nchunk, B), jnp.int32),
        pltpu.VMEM((nchunk, B), jnp.int32),
        pltpu.VMEM((2, B, d), F32),
        pltpu.VMEM((2, EB, 128), F32),
        pltpu.VMEM((2, B, d), F32),
        pltpu.VMEM_SHARED((N, d), F32),
    ]
    if SP:
        scratch.append(pltpu.VMEM_SHARED((N, d), F32))
    scratch += [pltpu.SemaphoreType.DMA] * 6

    @functools.partial(
        pl.kernel,
        out_type=jax.ShapeDtypeStruct((NC, N, d), F32),
        mesh=mesh,
        compiler_params=pltpu.CompilerParams(use_tc_tiling_on_sc=False),
        scratch_types=scratch,
    )
    def msg_pass(h_hbm, e_hbm, src_hbm, dst_hbm, zr_hbm, out_hbm, *scr):
        srcs_v, dsts_v, hs_v, e_v, m_v, acc_sh = scr[:6]
        h_src = scr[6] if SP else h_hbm
        gsem0, gsem1, esem0, esem1, ssem0, ssem1 = scr[7 if SP else 6:]
        c = lax.axis_index("c")
        s = lax.axis_index("s")
        wid = c * NS + s
        gsem = (gsem0, gsem1)
        esem = (esem0, esem1)
        ssem = (ssem0, ssem1)

        nrc_mine = NRC // NS + jnp.where(s < NRC % NS, 1, 0)

        def zero_chunk(k, carry):
            r0 = (s + k * NS) * RCH
            pltpu.sync_copy(zr_hbm, acc_sh.at[pl.ds(r0, RCH), :])
            if SP:
                pltpu.sync_copy(h_hbm.at[pl.ds(r0, RCH), :],
                                h_src.at[pl.ds(r0, RCH), :])
            return carry

        lax.fori_loop(0, nrc_mine, zero_chunk, 0)

        pltpu.sync_copy(src_hbm.at[wid], srcs_v)
        pltpu.sync_copy(dst_hbm.at[wid], dsts_v)
        plsc.subcore_barrier()

        def start(j, b):
            pltpu.async_copy(h_src.at[srcs_v.at[j]], hs_v.at[b], gsem[b])
            pltpu.async_copy(e_hbm.at[pl.ds(wid * ERW + j * EB, EB), :],
                             e_v.at[b], esem[b])

        def wait_in(j, b):
            pltpu.make_async_copy(h_src.at[srcs_v.at[j]], hs_v.at[b],
                                  gsem[b]).wait()
            pltpu.make_async_copy(e_hbm.at[pl.ds(wid * ERW + j * EB, EB), :],
                                  e_v.at[b], esem[b]).wait()

        def wait_scat(j, b):
            pltpu.make_async_copy(m_v.at[b], acc_sh.at[dsts_v.at[j]],
                                  ssem[b]).wait()

        def compute(b):
            def rows(r0, rcarry):
                er0 = r0 * (RU * d // 128)
                for rr in range(RU):
                    r = r0 * RU + rr
                    er = er0 + (rr * d) // 128
                    el0 = (rr * d) % 128
                    for cc in range(d // LANES):
                        sl = pl.ds(cc * LANES, LANES)
                        sle = pl.ds(el0 + cc * LANES, LANES)
                        m_v[b, r, sl] = jnp.maximum(
                            hs_v[b, r, sl] + e_v[b, er, sle], 0.0)
                return rcarry

            lax.fori_loop(0, B // RU, rows, 0)

        def step(j, b):
            wait_in(j, b)

            @pl.when(j >= 2)
            def _():
                wait_scat(j - 2, b)

            compute(b)

            @pl.when(j + 2 < nchunk)
            def _():
                start(j + 2, b)

            pltpu.async_copy(m_v.at[b], acc_sh.at[dsts_v.at[j]], ssem[b],
                             add=True)

        start(0, 0)
        if nchunk > 1:
            start(1, 1)

        def pair(i, carry):
            step(2 * i, 0)
            step(2 * i + 1, 1)
            return carry

        lax.fori_loop(0, nchunk // 2, pair, 0)
        if nchunk % 2:
            step(nchunk - 1, (nchunk - 1) % 2)
        if nchunk >= 2:
            wait_scat(nchunk - 2, (nchunk - 2) % 2)
        wait_scat(nchunk - 1, (nchunk - 1) % 2)
        plsc.subcore_barrier()

        def out_chunk(k, carry):
            r0 = (s + k * NS) * RCH
            pltpu.sync_copy(acc_sh.at[pl.ds(r0, RCH), :],
                            out_hbm.at[c, pl.ds(r0, RCH), :])
            return carry

        lax.fori_loop(0, nrc_mine, out_chunk, 0)

    return msg_pass


_DNT = (((0,), (0,)), ((), ()))


def _edge_transform_e0(ea_t, We0, be0, D):
    DE, E = ea_t.shape
    BE = 3200
    nb = E // BE

    def body(a_ref, W0_ref, b0_ref, e0_ref):
        a = a_ref[...]
        e0_ref[...] = lax.dot_general(
            a, W0_ref[...], _DNT, preferred_element_type=F32) + b0_ref[...]

    full = lambda shape: pl.BlockSpec(shape, lambda i: (0, 0))
    return pl.pallas_call(
        body,
        grid=(nb,),
        in_specs=[
            pl.BlockSpec((DE, BE), lambda i: (0, i)),
            full((DE, D)), full((1, D)),
        ],
        out_specs=pl.BlockSpec((BE, D), lambda i: (i, 0)),
        out_shape=jax.ShapeDtypeStruct((E, D), F32),
    )(ea_t, We0, be0)


def _edge_transform_e12(ea_t, We1, be1, We2, be2):
    DE, E = ea_t.shape
    BE = 3200
    nb = E // BE
    R = BE * H // 128

    def body(a_ref, W1_ref, b1_ref, W2_ref, b2_ref, e1_ref, e2_ref):
        a = a_ref[...]
        r1 = lax.dot_general(
            a, W1_ref[...], _DNT, preferred_element_type=F32) + b1_ref[...]
        r2 = lax.dot_general(
            a, W2_ref[...], _DNT, preferred_element_type=F32) + b2_ref[...]
        e1_ref[...] = jnp.concatenate(
            [r1[i * R:(i + 1) * R, :] for i in range(128 // H)], axis=1)
        e2_ref[...] = jnp.concatenate(
            [r2[i * R:(i + 1) * R, :] for i in range(128 // H)], axis=1)

    full = lambda shape: pl.BlockSpec(shape, lambda i: (0, 0))
    return pl.pallas_call(
        body,
        grid=(nb,),
        in_specs=[
            pl.BlockSpec((DE, BE), lambda i: (0, i)),
            full((DE, H)), full((1, H)),
            full((DE, H)), full((1, H)),
        ],
        out_specs=[
            pl.BlockSpec((R, 128), lambda i: (i, 0)),
            pl.BlockSpec((R, 128), lambda i: (i, 0)),
        ],
        out_shape=[
            jax.ShapeDtypeStruct((E * H // 128, 128), F32),
            jax.ShapeDtypeStruct((E * H // 128, 128), F32),
        ],
    )(ea_t, We1, be1, We2, be2)


def _node_mlp(h, p0, p1, W1a, b1a, W1b, b1b):
    N, d = h.shape
    BN = 2000
    nb = N // BN

    def body(h_ref, p0_ref, p1_ref, Wa_ref, ba_ref, Wb_ref, bb_ref, o_ref):
        z = h_ref[...] + p0_ref[...] + p1_ref[...]
        z1 = jnp.maximum(
            jnp.dot(z, Wa_ref[...], preferred_element_type=F32) + ba_ref[...], 0.0)
        z2 = jnp.dot(z1, Wb_ref[...], preferred_element_type=F32) + bb_ref[...]
        o_ref[...] = jnp.maximum(z2, 0.0)

    full = lambda shape: pl.BlockSpec(shape, lambda i: (0, 0))
    return pl.pallas_call(
        body,
        grid=(nb,),
        in_specs=[
            pl.BlockSpec((BN, d), lambda i: (i, 0)),
            pl.BlockSpec((BN, d), lambda i: (i, 0)),
            pl.BlockSpec((BN, d), lambda i: (i, 0)),
            full((d, H)), full((1, H)),
            full((H, H)), full((1, H)),
        ],
        out_specs=pl.BlockSpec((BN, H), lambda i: (i, 0)),
        out_shape=jax.ShapeDtypeStruct((N, H), F32),
    )(h, p0, p1, W1a, b1a, W1b, b1b)


def _final_stage(h1, h2, p0, p1, batch3, W1a2, b1a2, W1b2, b1b2,
                 WjkX, bjkX, Wc1p, bc1p, Wc2p, bc2p):
    N = h1.shape[0]
    BN = 2000
    nb = N // BN

    def body(h1_ref, h2_ref, p0_ref, p1_ref, b_ref, Wa_ref, ba_ref,
             Wb_ref, bb_ref, Wjk_ref, bjk_ref, Wc1_ref, bc1_ref,
             Wc2_ref, bc2_ref, o_ref, sums_scr):
        pid = pl.program_id(0)

        @pl.when(pid == 0)
        def _():
            sums_scr[...] = jnp.zeros((G, 64), F32)

        z = h2_ref[...] + p0_ref[...] + p1_ref[...]
        z1 = jnp.maximum(
            jnp.dot(z, Wa_ref[...], preferred_element_type=F32) + ba_ref[...], 0.0)
        z2 = jnp.dot(z1, Wb_ref[...], preferred_element_type=F32) + bb_ref[...]
        h3 = jnp.maximum(z2, 0.0)

        cat = jnp.concatenate([h1_ref[...], h2_ref[...], h3], axis=1)
        o = jnp.dot(cat, Wjk_ref[...], preferred_element_type=F32) + bjk_ref[...]

        b = b_ref[0]
        oh = (lax.broadcasted_iota(jnp.int32, (G, BN), 0) == b).astype(F32)
        sums_scr[...] += jnp.dot(oh, o, preferred_element_type=F32)

        @pl.when(pid == nb - 1)
        def _():
            se = sums_scr[...]
            pooled = se[:, :H] / jnp.maximum(se[:, H:H + 1], 1.0)
            t = jnp.maximum(
                jnp.dot(pooled, Wc1_ref[...], preferred_element_type=F32)
                + bc1_ref[...], 0.0)
            logits = (jnp.dot(t, Wc2_ref[...], preferred_element_type=F32)
                      + bc2_ref[...])
            maskb = jnp.where(
                lax.broadcasted_iota(jnp.int32, (G, 128), 1) < CLS, 0.0, -1e30)
            logits = logits + maskb
            m = jnp.max(logits, axis=1, keepdims=True)
            lse = m + jnp.log(jnp.sum(jnp.exp(logits - m), axis=1, keepdims=True))
            o_ref[...] = logits - lse

    full = lambda shape: pl.BlockSpec(shape, lambda i: tuple(0 for _ in shape))
    return pl.pallas_call(
        body,
        grid=(nb,),
        in_specs=[
            pl.BlockSpec((BN, H), lambda i: (i, 0)),
            pl.BlockSpec((BN, H), lambda i: (i, 0)),
            pl.BlockSpec((BN, H), lambda i: (i, 0)),
            pl.BlockSpec((BN, H), lambda i: (i, 0)),
            pl.BlockSpec((1, 1, BN), lambda i: (i, 0, 0)),
            full((H, H)), full((1, H)),
            full((H, H)), full((1, H)),
            full((3 * H, 64)), full((1, 64)),
            full((H, 128)), full((1, 128)),
            full((128, 128)), full((1, 128)),
        ],
        out_specs=pl.BlockSpec((G, 128), lambda i: (0, 0)),
        out_shape=jax.ShapeDtypeStruct((G, 128), F32),
        scratch_shapes=[pltpu.VMEM((G, 64), F32)],
    )(h1, h2, p0, p1, batch3, W1a2, b1a2, W1b2, b1b2,
      WjkX, bjkX, Wc1p, bc1p, Wc2p, bc2p)


def kernel(x, edge_index, edge_attr, batch,
           We0, be0, W1a0, b1a0, W1b0, b1b0,
           We1, be1, W1a1, b1a1, W1b1, b1b1,
           We2, be2, W1a2, b1a2, W1b2, b1b2,
           Wjk, bjk, Wc1, bc1, Wc2, bc2):
    N, D = x.shape
    E = edge_index.shape[1]
    B0, B1 = 40, 80
    src0 = edge_index[0].reshape(NW, E // (NW * B0), B0)
    dst0 = edge_index[1].reshape(NW, E // (NW * B0), B0)

    BE, R = 3200, 3200 * H // 128
    idx = jnp.arange(E, dtype=jnp.int32)
    q, jj = idx // 4, idx % 4
    orig = (q // R) * BE + jj * R + (q % R)
    src1 = edge_index[0][orig].reshape(NW, E // (NW * B1), B1)
    dst1 = edge_index[1][orig].reshape(NW, E // (NW * B1), B1)

    r1 = lambda v: v.reshape(1, -1)
    ea_t = edge_attr.T

    e0 = _edge_transform_e0(ea_t, We0, r1(be0), D)

    zr_d = jnp.zeros((200, D), F32)
    zr_h = jnp.zeros((200, H), F32)

    p = _make_msg_pass(N, D, E, B0)(x, e0, src0, dst0, zr_d)
    e1r, e2r = _edge_transform_e12(ea_t, We1, r1(be1), We2, r1(be2))
    h1 = _node_mlp(x, p[0], p[1], W1a0, r1(b1a0), W1b0, r1(b1b0))

    mp32 = _make_msg_pass(N, H, E, B1)
    p = mp32(h1, e1r, src1, dst1, zr_h)
    h2 = _node_mlp(h1, p[0], p[1], W1a1, r1(b1a1), W1b1, r1(b1b1))

    p = mp32(h2, e2r, src1, dst1, zr_h)

    WjkX = jnp.zeros((3 * H, 64), F32).at[:, :H].set(Wjk)
    bjkX = jnp.zeros((1, 64), F32).at[0, :H].set(bjk).at[0, H].set(1.0)
    Wc1p = jnp.zeros((H, 128), F32).at[:, :Wc1.shape[1]].set(Wc1)
    bc1p = jnp.zeros((1, 128), F32).at[0, :bc1.shape[0]].set(bc1)
    Wc2p = jnp.zeros((128, 128), F32).at[:Wc2.shape[0], :CLS].set(Wc2)
    bc2p = jnp.zeros((1, 128), F32).at[0, :CLS].set(bc2)

    BN = 2000
    batch3 = batch.reshape(N // BN, 1, BN)

    outp = _final_stage(h1, h2, p[0], p[1], batch3,
                        W1a2, r1(b1a2), W1b2, r1(b1b2),
                        WjkX, bjkX, Wc1p, bc1p, Wc2p, bc2p)
    return outp[:, :CLS]
